# Initial kernel scaffold; baseline (speedup 1.0000x reference)
import jax
import jax.numpy as jnp
from jax.experimental import pallas as pl


def kernel(x, edge_index, W0, b0, W1, b1, g0, be0, g1, be1, cW1, cb1, cW2, cb2):
    raise NotImplementedError("write your pallas kernel here")


# speedup vs baseline: 4.7254x; 4.7254x over previous
import dataclasses
import functools

import jax
import jax.numpy as jnp
from jax import lax
from jax.experimental import pallas as pl
from jax.experimental.pallas import tpu as pltpu
from jax.experimental.pallas import tpu_sc as plsc

N = 10000
E = 320000
D = 128
NC, NS = 2, 16
NW = NC * NS
CH = 128
NCHUNK = 80
EPT = CH * NCHUNK
E2 = EPT * NW
TRASH = 10016
NP = 10240
RPT = NP // NS


def _zero16():
    return jnp.zeros((16,), jnp.float32)


def _agg_body(with_cnt, *refs):
    if with_cnt:
        (h_hbm, src_hbm, dst_hbm, out_hbm, cnt_hbm,
         srcv0, srcv1, dstv0, dstv1, rows0, rows1, cntbuf,
         acc, gsem0, gsem1) = refs
    else:
        (h_hbm, src_hbm, dst_hbm, out_hbm,
         srcv0, srcv1, dstv0, dstv1, rows0, rows1,
         acc, gsem0, gsem1) = refs
        cntbuf = cnt_hbm = None
    srcv = (srcv0, srcv1)
    dstv = (dstv0, dstv1)
    rows = (rows0, rows1)
    gsem = (gsem0, gsem1)
    cid = lax.axis_index("c")
    sid = lax.axis_index("s")
    wid = cid * NS + sid
    ebase = wid * EPT

    @pl.loop(0, CH)
    def _(r):
        @pl.loop(0, D, step=16)
        def _(c):
            rows0[r, pl.ds(c, 16)] = _zero16()

    if with_cnt:
        @pl.loop(0, NP, step=16)
        def _(j):
            cntbuf[pl.ds(j, 16)] = _zero16()

    for k in range(RPT // CH):
        pltpu.sync_copy(rows0, acc.at[pl.ds(sid * RPT + k * CH, CH)])

    plsc.subcore_barrier()

    ones16 = jnp.ones((16,), jnp.float32)

    def load_idx(i, b):
        pltpu.sync_copy(src_hbm.at[pl.ds(ebase + i * CH, CH)], srcv[b])
        pltpu.sync_copy(dst_hbm.at[pl.ds(ebase + i * CH, CH)], dstv[b])

    def issue_gather(b):
        pltpu.async_copy(h_hbm.at[srcv[b]], rows[b], gsem[b])

    def wait_gather(b):
        pltpu.make_async_copy(h_hbm.at[srcv[b]], rows[b], gsem[b]).wait()

    def consume(b):
        pltpu.sync_copy(rows[b], acc.at[dstv[b]], add=True)
        if with_cnt:
            @pl.loop(0, CH, step=16)
            def _(j):
                plsc.addupdate_scatter(cntbuf, [dstv[b][pl.ds(j, 16)]],
                                       ones16)

    load_idx(0, 0)
    load_idx(1, 1)
    issue_gather(0)

    @pl.loop(0, NCHUNK // 2 - 1)
    def _(g):
        i0 = g * 2
        for b in (0, 1):
            wait_gather(b)
            issue_gather(1 - b)
            consume(b)
            load_idx(i0 + b + 2, b)

    wait_gather(0)
    issue_gather(1)
    consume(0)
    wait_gather(1)
    consume(1)

    plsc.subcore_barrier()

    for k in range(RPT // CH):
        r0 = sid * RPT + k * CH
        pltpu.sync_copy(acc.at[pl.ds(r0, CH)], rows0)
        pltpu.sync_copy(rows0, out_hbm.at[cid, pl.ds(r0, CH)])
    if with_cnt:
        pltpu.sync_copy(cntbuf, cnt_hbm.at[cid, sid])


_SC_MESH = plsc.VectorSubcoreMesh(core_axis_name="c", subcore_axis_name="s")

_SC_CP = pltpu.CompilerParams()
if "needs_layout_passes" in pltpu.CompilerParams.__dataclass_fields__:
    _SC_CP = dataclasses.replace(_SC_CP, needs_layout_passes=False)


def _make_agg(with_cnt):
    out_type = [jax.ShapeDtypeStruct((NC, NP, D), jnp.float32)]
    scratch = [
        pltpu.VMEM((CH,), jnp.int32),
        pltpu.VMEM((CH,), jnp.int32),
        pltpu.VMEM((CH,), jnp.int32),
        pltpu.VMEM((CH,), jnp.int32),
        pltpu.VMEM((CH, D), jnp.float32),
        pltpu.VMEM((CH, D), jnp.float32),
    ]
    if with_cnt:
        out_type.append(jax.ShapeDtypeStruct((NC, NS, NP), jnp.float32))
        scratch.append(pltpu.VMEM((NP,), jnp.float32))
    scratch += [
        pltpu.VMEM_SHARED((NP, D), jnp.float32),
        pltpu.SemaphoreType.DMA,
        pltpu.SemaphoreType.DMA,
    ]
    return pl.kernel(
        functools.partial(_agg_body, with_cnt),
        out_type=tuple(out_type) if with_cnt else out_type[0],
        mesh=_SC_MESH,
        scratch_types=scratch,
        compiler_params=_SC_CP,
    )


_agg_cnt = _make_agg(True)
_agg = _make_agg(False)


def _lin0(x, W0, b0):
    def body(x_ref, w_ref, b_ref, o_ref):
        o_ref[...] = (
            jnp.dot(x_ref[...], w_ref[...],
                    preferred_element_type=jnp.float32) + b_ref[...]
        )
    return pl.pallas_call(
        body, out_shape=jax.ShapeDtypeStruct((N, D), jnp.float32),
    )(x, W0, b0)


def _mean_bn_relu(p_ref, h_ref, cp_ref, g, be):
    cnt = jnp.sum(cp_ref[...], axis=(0, 1))[:N] + 1.0
    t = (p_ref[0, :N] + p_ref[1, :N] + h_ref[...]) / cnt[:, None]
    mu = jnp.mean(t, axis=0)
    var = jnp.mean((t - mu) ** 2, axis=0)
    tn = (t - mu) * lax.rsqrt(var + 1e-5) * g + be
    return jnp.maximum(tn, 0.0)


def _mid(p, h0, cp, W1, b1, g0, be0):
    def body(p_ref, h_ref, cp_ref, w_ref, b_ref, g_ref, be_ref, o_ref):
        r = _mean_bn_relu(p_ref, h_ref, cp_ref, g_ref[...], be_ref[...])
        o_ref[...] = (
            jnp.dot(r, w_ref[...], preferred_element_type=jnp.float32)
            + b_ref[...]
        )
    return pl.pallas_call(
        body, out_shape=jax.ShapeDtypeStruct((N, D), jnp.float32),
    )(p, h0, cp, W1, b1, g0, be0)


def _head(q, h1, cp, g1, be1, cW1, cb1, cW2, cb2):
    def body(q_ref, h_ref, cp_ref, g_ref, be_ref, w1_ref, b1_ref,
             w2_ref, b2_ref, o_ref):
        r = _mean_bn_relu(q_ref, h_ref, cp_ref, g_ref[...], be_ref[...])
        z = jnp.maximum(
            jnp.dot(r, w1_ref[...], preferred_element_type=jnp.float32)
            + b1_ref[...], 0.0)
        o_ref[...] = (
            jnp.dot(z, w2_ref[...], preferred_element_type=jnp.float32)
            + b2_ref[...]
        )
    return pl.pallas_call(
        body, out_shape=jax.ShapeDtypeStruct((N, 2), jnp.float32),
    )(q, h1, cp, g1, be1, cW1, cb1, cW2, cb2)


def kernel(x, edge_index, W0, b0, W1, b1, g0, be0, g1, be1,
           cW1, cb1, cW2, cb2):
    ei = edge_index.astype(jnp.int32)
    pad = E2 - E
    src = jnp.concatenate([ei[0], jnp.zeros((pad,), jnp.int32)])
    dst = jnp.concatenate([ei[1], jnp.full((pad,), TRASH, jnp.int32)])
    h0 = _lin0(x, W0, b0)
    p, cp = _agg_cnt(h0, src, dst)
    h1 = _mid(p, h0, cp, W1, b1, g0, be0)
    q = _agg(h1, src, dst)
    return _head(q, h1, cp, g1, be1, cW1, cb1, cW2, cb2)


# speedup vs baseline: 13.8652x; 2.9342x over previous
import dataclasses
import functools

import jax
import jax.numpy as jnp
from jax import lax
from jax.experimental import pallas as pl
from jax.experimental.pallas import tpu as pltpu
from jax.experimental.pallas import tpu_sc as plsc

N = 10000
E = 320000
D = 128
NC, NS = 2, 16
NW = NC * NS
CH = 128
NCHUNK = 80
EPT = CH * NCHUNK
E2 = EPT * NW
TRASH = 10000
NP = 10240
RPT = NP // NS


def _zero16():
    return jnp.zeros((16,), jnp.float32)


def _agg_body(with_cnt, *refs):
    if with_cnt:
        (h_hbm, src_hbm, dst_hbm, out_hbm, cnt_hbm,
         srcv0, srcv1, dstv0, dstv1, rows0, rows1, cntbuf,
         acc, gsem0, gsem1) = refs
    else:
        (h_hbm, src_hbm, dst_hbm, out_hbm,
         srcv0, srcv1, dstv0, dstv1, rows0, rows1,
         acc, gsem0, gsem1) = refs
        cntbuf = cnt_hbm = None
    srcv = (srcv0, srcv1)
    dstv = (dstv0, dstv1)
    rows = (rows0, rows1)
    gsem = (gsem0, gsem1)
    cid = lax.axis_index("c")
    sid = lax.axis_index("s")
    wid = cid * NS + sid
    ebase = wid * EPT

    @pl.loop(0, CH)
    def _(r):
        @pl.loop(0, D, step=16)
        def _(c):
            rows0[r, pl.ds(c, 16)] = _zero16()

    if with_cnt:
        @pl.loop(0, NP, step=16)
        def _(j):
            cntbuf[pl.ds(j, 16)] = _zero16()

    for k in range(RPT // CH):
        pltpu.sync_copy(rows0, acc.at[pl.ds(sid * RPT + k * CH, CH)])

    plsc.subcore_barrier()

    ones16 = jnp.ones((16,), jnp.float32)

    def load_idx(i, b):
        pltpu.sync_copy(src_hbm.at[pl.ds(ebase + i * CH, CH)], srcv[b])
        pltpu.sync_copy(dst_hbm.at[pl.ds(ebase + i * CH, CH)], dstv[b])

    def issue_gather(b):
        pltpu.async_copy(h_hbm.at[srcv[b]], rows[b], gsem[b])

    def wait_gather(b):
        pltpu.make_async_copy(h_hbm.at[srcv[b]], rows[b], gsem[b]).wait()

    def consume(b):
        pltpu.sync_copy(rows[b], acc.at[dstv[b]], add=True)
        if with_cnt:
            @pl.loop(0, CH, step=16)
            def _(j):
                plsc.addupdate_scatter(cntbuf, [dstv[b][pl.ds(j, 16)]],
                                       ones16)

    load_idx(0, 0)
    load_idx(1, 1)
    issue_gather(0)

    @pl.loop(0, NCHUNK // 2 - 1)
    def _(g):
        i0 = g * 2
        for b in (0, 1):
            wait_gather(b)
            issue_gather(1 - b)
            consume(b)
            load_idx(i0 + b + 2, b)

    wait_gather(0)
    issue_gather(1)
    consume(0)
    wait_gather(1)
    consume(1)

    plsc.subcore_barrier()

    for k in range(RPT // CH):
        r0 = sid * RPT + k * CH
        pltpu.sync_copy(acc.at[pl.ds(r0, CH)], rows0)
        pltpu.sync_copy(rows0, out_hbm.at[cid, pl.ds(r0, CH)])
    if with_cnt:
        pltpu.sync_copy(cntbuf, cnt_hbm.at[cid, sid])


_SC_MESH = plsc.VectorSubcoreMesh(core_axis_name="c", subcore_axis_name="s")

_SC_CP = pltpu.CompilerParams()
if "needs_layout_passes" in pltpu.CompilerParams.__dataclass_fields__:
    _SC_CP = dataclasses.replace(_SC_CP, needs_layout_passes=False)


def _make_agg(with_cnt):
    out_type = [jax.ShapeDtypeStruct((NC, NP, D), jnp.float32)]
    scratch = [
        pltpu.VMEM((CH,), jnp.int32),
        pltpu.VMEM((CH,), jnp.int32),
        pltpu.VMEM((CH,), jnp.int32),
        pltpu.VMEM((CH,), jnp.int32),
        pltpu.VMEM((CH, D), jnp.float32),
        pltpu.VMEM((CH, D), jnp.float32),
    ]
    if with_cnt:
        out_type.append(jax.ShapeDtypeStruct((NC, NS, NP), jnp.float32))
        scratch.append(pltpu.VMEM((NP,), jnp.float32))
    scratch += [
        pltpu.VMEM_SHARED((NP, D), jnp.float32),
        pltpu.SemaphoreType.DMA,
        pltpu.SemaphoreType.DMA,
    ]
    return pl.kernel(
        functools.partial(_agg_body, with_cnt),
        out_type=tuple(out_type) if with_cnt else out_type[0],
        mesh=_SC_MESH,
        scratch_types=scratch,
        compiler_params=_SC_CP,
    )


_agg_cnt = _make_agg(True)
_agg = _make_agg(False)


def _lin0(x, W0, b0):
    def body(x_ref, w_ref, b_ref, o_ref):
        o_ref[...] = (
            jnp.dot(x_ref[...], w_ref[...],
                    preferred_element_type=jnp.float32) + b_ref[...]
        )
    return pl.pallas_call(
        body, out_shape=jax.ShapeDtypeStruct((N, D), jnp.float32),
    )(x, W0, b0)


def _mean_bn_relu(p_ref, h_ref, cp_ref, g, be):
    cnt = jnp.sum(cp_ref[...], axis=(0, 1))[:N] + 1.0
    t = (p_ref[0, :N] + p_ref[1, :N] + h_ref[...]) / cnt[:, None]
    mu = jnp.mean(t, axis=0)
    var = jnp.mean((t - mu) ** 2, axis=0)
    tn = (t - mu) * lax.rsqrt(var + 1e-5) * g + be
    return jnp.maximum(tn, 0.0)


def _mid(p, h0, cp, W1, b1, g0, be0):
    def body(p_ref, h_ref, cp_ref, w_ref, b_ref, g_ref, be_ref, o_ref):
        r = _mean_bn_relu(p_ref, h_ref, cp_ref, g_ref[...], be_ref[...])
        o_ref[...] = (
            jnp.dot(r, w_ref[...], preferred_element_type=jnp.float32)
            + b_ref[...]
        )
    return pl.pallas_call(
        body, out_shape=jax.ShapeDtypeStruct((N, D), jnp.float32),
    )(p, h0, cp, W1, b1, g0, be0)


def _head(q, h1, cp, g1, be1, cW1, cb1, cW2, cb2):
    def body(q_ref, h_ref, cp_ref, g_ref, be_ref, w1_ref, b1_ref,
             w2_ref, b2_ref, o_ref):
        r = _mean_bn_relu(q_ref, h_ref, cp_ref, g_ref[...], be_ref[...])
        z = jnp.maximum(
            jnp.dot(r, w1_ref[...], preferred_element_type=jnp.float32)
            + b1_ref[...], 0.0)
        o_ref[...] = (
            jnp.dot(z, w2_ref[...], preferred_element_type=jnp.float32)
            + b2_ref[...]
        )
    return pl.pallas_call(
        body, out_shape=jax.ShapeDtypeStruct((N, 2), jnp.float32),
    )(q, h1, cp, g1, be1, cW1, cb1, cW2, cb2)


def kernel(x, edge_index, W0, b0, W1, b1, g0, be0, g1, be1,
           cW1, cb1, cW2, cb2):
    ei = edge_index.astype(jnp.int32)
    pad = E2 - E
    pidx = lax.iota(jnp.int32, pad)
    src = jnp.concatenate([ei[0], pidx % N])
    dst = jnp.concatenate([ei[1], TRASH + pidx % (NP - TRASH)])
    h0 = _lin0(x, W0, b0)
    p, cp = _agg_cnt(h0, src, dst)
    h1 = _mid(p, h0, cp, W1, b1, g0, be0)
    q = _agg(h1, src, dst)
    return _head(q, h1, cp, g1, be1, cW1, cb1, cW2, cb2)


# speedup vs baseline: 15.6936x; 1.1319x over previous
import dataclasses
import functools

import jax
import jax.numpy as jnp
from jax import lax
from jax.experimental import pallas as pl
from jax.experimental.pallas import tpu as pltpu
from jax.experimental.pallas import tpu_sc as plsc

N = 10000
E = 320000
D = 128
NC, NS = 2, 16
NW = NC * NS
CH = 128
NCHUNK = 80
EPT = CH * NCHUNK
E2 = EPT * NW
TRASH = 10000
NP = 10240
RPT = NP // NS


def _zero16():
    return jnp.zeros((16,), jnp.float32)


def _agg_body(with_cnt, *refs):
    if with_cnt:
        (h_hbm, src_hbm, dst_hbm, out_hbm, cnt_hbm,
         srcv0, srcv1, dstv0, dstv1, rows0, rows1, cntbuf,
         acc, gsem0, gsem1, isem0, isem1) = refs
    else:
        (h_hbm, src_hbm, dst_hbm, out_hbm,
         srcv0, srcv1, dstv0, dstv1, rows0, rows1,
         acc, gsem0, gsem1, isem0, isem1) = refs
        cntbuf = cnt_hbm = None
    srcv = (srcv0, srcv1)
    dstv = (dstv0, dstv1)
    rows = (rows0, rows1)
    gsem = (gsem0, gsem1)
    isem = (isem0, isem1)
    cid = lax.axis_index("c")
    sid = lax.axis_index("s")
    wid = cid * NS + sid
    ebase = wid * EPT

    @pl.loop(0, CH)
    def _(r):
        @pl.loop(0, D, step=16)
        def _(c):
            rows0[r, pl.ds(c, 16)] = _zero16()

    if with_cnt:
        @pl.loop(0, NP, step=16)
        def _(j):
            cntbuf[pl.ds(j, 16)] = _zero16()

    for k in range(RPT // CH):
        pltpu.sync_copy(rows0, acc.at[pl.ds(sid * RPT + k * CH, CH)])

    plsc.subcore_barrier()

    ones16 = jnp.ones((16,), jnp.float32)

    def load_idx(i, b):
        pltpu.async_copy(src_hbm.at[pl.ds(ebase + i * CH, CH)], srcv[b],
                         isem[b])
        pltpu.async_copy(dst_hbm.at[pl.ds(ebase + i * CH, CH)], dstv[b],
                         isem[b])

    def wait_idx(i, b):
        pltpu.make_async_copy(src_hbm.at[pl.ds(ebase + i * CH, CH)],
                              srcv[b], isem[b]).wait()
        pltpu.make_async_copy(dst_hbm.at[pl.ds(ebase + i * CH, CH)],
                              dstv[b], isem[b]).wait()

    def issue_gather(b):
        pltpu.async_copy(h_hbm.at[srcv[b]], rows[b], gsem[b])

    def wait_gather(b):
        pltpu.make_async_copy(h_hbm.at[srcv[b]], rows[b], gsem[b]).wait()

    def consume(b):
        pltpu.sync_copy(rows[b], acc.at[dstv[b]], add=True)
        if with_cnt:
            @pl.loop(0, CH, step=16)
            def _(j):
                plsc.addupdate_scatter(cntbuf, [dstv[b][pl.ds(j, 16)]],
                                       ones16)

    load_idx(0, 0)
    load_idx(1, 1)
    wait_idx(0, 0)
    issue_gather(0)

    @pl.loop(0, NCHUNK // 2 - 1)
    def _(g):
        i0 = g * 2
        for b in (0, 1):
            wait_gather(b)
            wait_idx(i0 + b + 1, 1 - b)
            issue_gather(1 - b)
            consume(b)
            load_idx(i0 + b + 2, b)

    wait_gather(0)
    wait_idx(NCHUNK - 1, 1)
    issue_gather(1)
    consume(0)
    wait_gather(1)
    consume(1)

    plsc.subcore_barrier()

    for k in range(RPT // CH):
        r0 = sid * RPT + k * CH
        pltpu.sync_copy(acc.at[pl.ds(r0, CH)], rows0)
        pltpu.sync_copy(rows0, out_hbm.at[cid, pl.ds(r0, CH)])
    if with_cnt:
        pltpu.sync_copy(cntbuf, cnt_hbm.at[cid, sid])


_SC_MESH = plsc.VectorSubcoreMesh(core_axis_name="c", subcore_axis_name="s")

_SC_CP = pltpu.CompilerParams()
if "needs_layout_passes" in pltpu.CompilerParams.__dataclass_fields__:
    _SC_CP = dataclasses.replace(_SC_CP, needs_layout_passes=False)


def _make_agg(with_cnt):
    out_type = [jax.ShapeDtypeStruct((NC, NP, D), jnp.float32)]
    scratch = [
        pltpu.VMEM((CH,), jnp.int32),
        pltpu.VMEM((CH,), jnp.int32),
        pltpu.VMEM((CH,), jnp.int32),
        pltpu.VMEM((CH,), jnp.int32),
        pltpu.VMEM((CH, D), jnp.float32),
        pltpu.VMEM((CH, D), jnp.float32),
    ]
    if with_cnt:
        out_type.append(jax.ShapeDtypeStruct((NC, NS, NP), jnp.float32))
        scratch.append(pltpu.VMEM((NP,), jnp.float32))
    scratch += [
        pltpu.VMEM_SHARED((NP, D), jnp.float32),
        pltpu.SemaphoreType.DMA,
        pltpu.SemaphoreType.DMA,
        pltpu.SemaphoreType.DMA,
        pltpu.SemaphoreType.DMA,
    ]
    return pl.kernel(
        functools.partial(_agg_body, with_cnt),
        out_type=tuple(out_type) if with_cnt else out_type[0],
        mesh=_SC_MESH,
        scratch_types=scratch,
        compiler_params=_SC_CP,
    )


_agg_cnt = _make_agg(True)
_agg = _make_agg(False)


def _lin0(x, W0, b0):
    def body(x_ref, w_ref, b_ref, o_ref):
        o_ref[...] = (
            jnp.dot(x_ref[...], w_ref[...],
                    preferred_element_type=jnp.float32) + b_ref[...]
        )
    return pl.pallas_call(
        body, out_shape=jax.ShapeDtypeStruct((N, D), jnp.float32),
    )(x, W0, b0)


def _mean_bn_relu(p_ref, h_ref, cp_ref, g, be):
    cnt = jnp.sum(cp_ref[...], axis=(0, 1))[:N] + 1.0
    t = (p_ref[0, :N] + p_ref[1, :N] + h_ref[...]) / cnt[:, None]
    mu = jnp.mean(t, axis=0)
    var = jnp.mean((t - mu) ** 2, axis=0)
    tn = (t - mu) * lax.rsqrt(var + 1e-5) * g + be
    return jnp.maximum(tn, 0.0)


def _mid(p, h0, cp, W1, b1, g0, be0):
    def body(p_ref, h_ref, cp_ref, w_ref, b_ref, g_ref, be_ref, o_ref):
        r = _mean_bn_relu(p_ref, h_ref, cp_ref, g_ref[...], be_ref[...])
        o_ref[...] = (
            jnp.dot(r, w_ref[...], preferred_element_type=jnp.float32)
            + b_ref[...]
        )
    return pl.pallas_call(
        body, out_shape=jax.ShapeDtypeStruct((N, D), jnp.float32),
    )(p, h0, cp, W1, b1, g0, be0)


def _head(q, h1, cp, g1, be1, cW1, cb1, cW2, cb2):
    def body(q_ref, h_ref, cp_ref, g_ref, be_ref, w1_ref, b1_ref,
             w2_ref, b2_ref, o_ref):
        r = _mean_bn_relu(q_ref, h_ref, cp_ref, g_ref[...], be_ref[...])
        z = jnp.maximum(
            jnp.dot(r, w1_ref[...], preferred_element_type=jnp.float32)
            + b1_ref[...], 0.0)
        o_ref[...] = (
            jnp.dot(z, w2_ref[...], preferred_element_type=jnp.float32)
            + b2_ref[...]
        )
    return pl.pallas_call(
        body, out_shape=jax.ShapeDtypeStruct((N, 2), jnp.float32),
    )(q, h1, cp, g1, be1, cW1, cb1, cW2, cb2)


def kernel(x, edge_index, W0, b0, W1, b1, g0, be0, g1, be1,
           cW1, cb1, cW2, cb2):
    ei = edge_index.astype(jnp.int32)
    pad = E2 - E
    pidx = lax.iota(jnp.int32, pad)
    src = jnp.concatenate([ei[0], pidx % N])
    dst = jnp.concatenate([ei[1], TRASH + pidx % (NP - TRASH)])
    h0 = _lin0(x, W0, b0)
    p, cp = _agg_cnt(h0, src, dst)
    h1 = _mid(p, h0, cp, W1, b1, g0, be0)
    q = _agg(h1, src, dst)
    return _head(q, h1, cp, g1, be1, cW1, cb1, cW2, cb2)
